# baseline (speedup 1.0000x reference)
import functools

import numpy as np
import jax
import jax.numpy as jnp
from jax import lax
from jax.experimental import pallas as pl
from jax.experimental.pallas import tpu as pltpu
from jax.experimental.pallas import tpu_sc as plsc

_B, _H, _W = 16, 480, 640
_GROUPS_PER_TEC = (_H // 2) // 16
_WORDS = _H * _W // 8

_WMAT = np.zeros((16, 8), np.float32)
_NSEL = np.zeros((16, 8), np.float32)
for _slot, (_J, _Ax, _Bx, _Ay, _By) in enumerate([
    (2.0, 225.0, 50625.0, 128.0, 16384.0),
    (1.0, 0.0, 0.0, 0.0, 0.0),
    (2.0, 0.0, 0.0, 0.0, 0.0),
]):
    _WMAT[5 * _slot + 0, _slot] = _Bx + _By
    _WMAT[5 * _slot + 1, _slot] = -2.0 * _Ax
    _WMAT[5 * _slot + 2, _slot] = _J
    _WMAT[5 * _slot + 3, _slot] = -2.0 * _Ay
    _WMAT[5 * _slot + 4, _slot] = _J
    _NSEL[5 * _slot + 0, _slot] = 1.0


def _pack_body(lab_ref, out_ref):
    w = lab_ref[0, :, 0, :]
    for j in range(1, 8):
        w = w | (lab_ref[0, :, j, :] << (4 * j))
    out_ref[0] = w


_pack_call = pl.pallas_call(
    _pack_body,
    grid=(_B,),
    in_specs=[pl.BlockSpec((1, _H // 8, 8, _W), lambda b: (b, 0, 0, 0))],
    out_specs=pl.BlockSpec((1, _H // 8, _W), lambda b: (b, 0, 0)),
    out_shape=jax.ShapeDtypeStruct((_B, _H // 8, _W), jnp.int32),
)


_sc_mesh = plsc.VectorSubcoreMesh(core_axis_name="c", subcore_axis_name="s")


@functools.partial(
    pl.kernel,
    mesh=_sc_mesh,
    compiler_params=pltpu.CompilerParams(needs_layout_passes=False),
    out_type=jax.ShapeDtypeStruct((32, 15, 16), jnp.float32),
    scratch_types=[
        pltpu.VMEM((_H // 8, _W), jnp.int32),
        pltpu.VMEM((9, 16), jnp.float32),
        pltpu.VMEM((15, 16), jnp.float32),
    ],
)
def _sc_moments(packed_hbm, hb_hbm, out_hbm, table_v, h_v, mom_v):
    cid = lax.axis_index("c")
    sid = lax.axis_index("s")
    wid = sid * 2 + cid
    batch = wid >> 1
    half = wid & 1

    pltpu.sync_copy(packed_hbm.at[batch], table_v)
    pltpu.sync_copy(hb_hbm.at[batch], h_v)

    h00 = h_v[0]
    h01 = h_v[1]
    h02 = h_v[2]
    h10 = h_v[3]
    h11 = h_v[4]
    h12 = h_v[5]
    h20 = h_v[6]
    h21 = h_v[7]
    h22 = h_v[8]

    iota16 = lax.iota(jnp.int32, 16)
    iotaf = iota16.astype(jnp.float32)
    z = jnp.zeros((16,), jnp.float32)

    accs = [z] * 15

    r_base = (half * (_H // 2)).astype(jnp.float32)
    zi = jnp.zeros((16,), jnp.int32)
    h00p = h00 + 0.5 * h20
    h10p = h10 + 0.5 * h20
    for g in range(_GROUPS_PER_TEC):
        rvf = r_base + (g * 16) + iotaf
        bd = h21 * rvf + h22
        bx = h01 * rvf + h02 + 0.5 * bd
        by = h11 * rvf + h12 + 0.5 * bd

        def body(ci, carry, bx=bx, by=by, bd=bd):
            a3, q3, a4, q4, a8, q8 = carry
            cf = ci.astype(jnp.float32)
            cfv = cf + z
            cf2v = cf * cf + z
            combo_v = (ci | (1 << 18)) + zi
            rcp = 1.0 / (h20 * cfv + bd)
            xi = jnp.clip((h00p * cfv + bx) * rcp, 0.5, _W - 0.5).astype(jnp.int32)
            yi = jnp.clip((h10p * cfv + by) * rcp, 0.5, _H - 0.5).astype(jnp.int32)
            shift = (yi & 7) << 2
            w = plsc.load_gather(table_v, [yi >> 3, xi])
            code = (w >> shift) & 15
            m3 = code == 3
            m4 = code == 4
            m8 = code == 8
            return (a3 + jnp.where(m3, combo_v, zi), q3 + jnp.where(m3, cf2v, z),
                    a4 + jnp.where(m4, combo_v, zi), q4 + jnp.where(m4, cf2v, z),
                    a8 + jnp.where(m8, combo_v, zi), q8 + jnp.where(m8, cf2v, z))

        a3, q3, a4, q4, a8, q8 = lax.fori_loop(
            0, _W, body, (zi, z, zi, z, zi, z), unroll=4)

        rvf2 = rvf * rvf
        for slot, (acc_i, sc2_) in enumerate(((a3, q3), (a4, q4), (a8, q8))):
            cnt = (acc_i >> 18).astype(jnp.float32)
            sc_ = (acc_i & ((1 << 18) - 1)).astype(jnp.float32)
            accs[5 * slot + 0] = accs[5 * slot + 0] + cnt
            accs[5 * slot + 1] = accs[5 * slot + 1] + rvf * cnt
            accs[5 * slot + 2] = accs[5 * slot + 2] + rvf2 * cnt
            accs[5 * slot + 3] = accs[5 * slot + 3] + sc_
            accs[5 * slot + 4] = accs[5 * slot + 4] + sc2_

    for i in range(15):
        mom_v[i] = accs[i]
    pltpu.sync_copy(mom_v, out_hbm.at[wid])


def _loss_body(m_ref, w_ref, ns_ref, out_ref):
    s = (m_ref[:, 0] + m_ref[:, 1]).sum(axis=2)
    lin = jnp.dot(s, w_ref[...], preferred_element_type=jnp.float32)
    nsel = jnp.dot(s, ns_ref[...], preferred_element_type=jnp.float32)
    out_ref[...] = jnp.sum(lin / jnp.maximum(1.0, nsel)).reshape(1, 1)


_loss_call = pl.pallas_call(
    _loss_body,
    out_shape=jax.ShapeDtypeStruct((1, 1), jnp.float32),
)


def kernel(input_label, h, device=0):
    lab4 = input_label.reshape(_B, _H // 8, 8, _W)
    packed = _pack_call(lab4)
    hb = jnp.broadcast_to(
        h.astype(jnp.float32).reshape(_B, 9, 1), (_B, 9, 16))
    mom = _sc_moments(packed, hb)
    m4d = mom.reshape(_B, 2, 15, 16)
    m4d = jnp.pad(m4d, ((0, 0), (0, 0), (0, 1), (0, 0)))
    loss = _loss_call(m4d, jnp.asarray(_WMAT), jnp.asarray(_NSEL))
    return loss[0, 0]


# speedup vs baseline: 1.0862x; 1.0862x over previous
import functools

import numpy as np
import jax
import jax.numpy as jnp
from jax import lax
from jax.experimental import pallas as pl
from jax.experimental.pallas import tpu as pltpu
from jax.experimental.pallas import tpu_sc as plsc

_B, _H, _W = 16, 480, 640
_GROUPS_PER_TEC = (_H // 2) // 16
_WORDS = _H * _W // 8

_WMAT = np.zeros((16, 8), np.float32)
_NSEL = np.zeros((16, 8), np.float32)
for _slot, (_J, _Ax, _Bx, _Ay, _By) in enumerate([
    (2.0, 225.0, 50625.0, 128.0, 16384.0),
    (1.0, 0.0, 0.0, 0.0, 0.0),
    (2.0, 0.0, 0.0, 0.0, 0.0),
]):
    _WMAT[5 * _slot + 0, _slot] = _Bx + _By
    _WMAT[5 * _slot + 1, _slot] = -2.0 * _Ax
    _WMAT[5 * _slot + 2, _slot] = _J
    _WMAT[5 * _slot + 3, _slot] = -2.0 * _Ay
    _WMAT[5 * _slot + 4, _slot] = _J
    _NSEL[5 * _slot + 0, _slot] = 1.0


def _pack_body(lab_ref, out_ref):
    w = lab_ref[0, :, 0, :]
    for j in range(1, 8):
        w = w | (lab_ref[0, :, j, :] << (4 * j))
    out_ref[0] = w


_pack_call = pl.pallas_call(
    _pack_body,
    grid=(_B,),
    in_specs=[pl.BlockSpec((1, _H // 8, 8, _W), lambda b: (b, 0, 0, 0))],
    out_specs=pl.BlockSpec((1, _H // 8, _W), lambda b: (b, 0, 0)),
    out_shape=jax.ShapeDtypeStruct((_B, _H // 8, _W), jnp.int32),
)


_sc_mesh = plsc.VectorSubcoreMesh(core_axis_name="c", subcore_axis_name="s")


@functools.partial(
    pl.kernel,
    mesh=_sc_mesh,
    compiler_params=pltpu.CompilerParams(needs_layout_passes=False),
    out_type=jax.ShapeDtypeStruct((32, 15, 16), jnp.float32),
    scratch_types=[
        pltpu.VMEM((_WORDS,), jnp.int32),
        pltpu.VMEM((9, 16), jnp.float32),
        pltpu.VMEM((15, 16), jnp.float32),
    ],
)
def _sc_moments(packed_hbm, hb_hbm, out_hbm, table_v, h_v, mom_v):
    cid = lax.axis_index("c")
    sid = lax.axis_index("s")
    wid = sid * 2 + cid
    batch = wid >> 1
    half = wid & 1

    pltpu.sync_copy(packed_hbm.at[batch], table_v)
    pltpu.sync_copy(hb_hbm.at[batch], h_v)

    h00 = h_v[0]
    h01 = h_v[1]
    h02 = h_v[2]
    h10 = h_v[3]
    h11 = h_v[4]
    h12 = h_v[5]
    h20 = h_v[6]
    h21 = h_v[7]
    h22 = h_v[8]

    iota16 = lax.iota(jnp.int32, 16)
    iotaf = iota16.astype(jnp.float32)
    z = jnp.zeros((16,), jnp.float32)

    accs = [z] * 15

    r_base = (half * (_H // 2)).astype(jnp.float32)
    zi = jnp.zeros((16,), jnp.int32)
    h00p = h00 + 0.5 * h20
    h10p = h10 + 0.5 * h20
    for g in range(_GROUPS_PER_TEC):
        rvf = r_base + (g * 16) + iotaf
        bd = h21 * rvf + h22
        bx = h01 * rvf + h02 + 0.5 * bd
        by = h11 * rvf + h12 + 0.5 * bd

        def body(ci, carry, bx=bx, by=by, bd=bd):
            a3, q3, a4, q4, a8, q8 = carry
            cf = ci.astype(jnp.float32)
            cfv = cf + z
            cf2v = cf * cf + z
            combo_v = (ci | (1 << 18)) + zi
            rcp = 1.0 / (h20 * cfv + bd)
            xi = jnp.clip((h00p * cfv + bx) * rcp, 0.5, _W - 0.5).astype(jnp.int32)
            yi = jnp.clip((h10p * cfv + by) * rcp, 0.5, _H - 0.5).astype(jnp.int32)
            lin = (yi >> 3) * _W + xi
            shift = (yi & 7) << 2
            w = plsc.load_gather(table_v, [lin])
            code = (w >> shift) & 15
            m3 = code == 3
            m4 = code == 4
            m8 = code == 8
            return (a3 + jnp.where(m3, combo_v, zi), q3 + jnp.where(m3, cf2v, z),
                    a4 + jnp.where(m4, combo_v, zi), q4 + jnp.where(m4, cf2v, z),
                    a8 + jnp.where(m8, combo_v, zi), q8 + jnp.where(m8, cf2v, z))

        a3, q3, a4, q4, a8, q8 = lax.fori_loop(
            0, _W, body, (zi, z, zi, z, zi, z), unroll=4)

        rvf2 = rvf * rvf
        for slot, (acc_i, sc2_) in enumerate(((a3, q3), (a4, q4), (a8, q8))):
            cnt = (acc_i >> 18).astype(jnp.float32)
            sc_ = (acc_i & ((1 << 18) - 1)).astype(jnp.float32)
            accs[5 * slot + 0] = accs[5 * slot + 0] + cnt
            accs[5 * slot + 1] = accs[5 * slot + 1] + rvf * cnt
            accs[5 * slot + 2] = accs[5 * slot + 2] + rvf2 * cnt
            accs[5 * slot + 3] = accs[5 * slot + 3] + sc_
            accs[5 * slot + 4] = accs[5 * slot + 4] + sc2_

    for i in range(15):
        mom_v[i] = accs[i]
    pltpu.sync_copy(mom_v, out_hbm.at[wid])


def _loss_body(m_ref, w_ref, ns_ref, out_ref):
    s = (m_ref[:, 0] + m_ref[:, 1]).sum(axis=2)
    lin = jnp.dot(s, w_ref[...], preferred_element_type=jnp.float32)
    nsel = jnp.dot(s, ns_ref[...], preferred_element_type=jnp.float32)
    out_ref[...] = jnp.sum(lin / jnp.maximum(1.0, nsel)).reshape(1, 1)


_loss_call = pl.pallas_call(
    _loss_body,
    out_shape=jax.ShapeDtypeStruct((1, 1), jnp.float32),
)


def kernel(input_label, h, device=0):
    lab4 = input_label.reshape(_B, _H // 8, 8, _W)
    packed = _pack_call(lab4).reshape(_B, _WORDS)
    hb = jnp.broadcast_to(
        h.astype(jnp.float32).reshape(_B, 9, 1), (_B, 9, 16))
    mom = _sc_moments(packed, hb)
    m4d = mom.reshape(_B, 2, 15, 16)
    m4d = jnp.pad(m4d, ((0, 0), (0, 0), (0, 1), (0, 0)))
    loss = _loss_call(m4d, jnp.asarray(_WMAT), jnp.asarray(_NSEL))
    return loss[0, 0]
